# baseline (speedup 1.0000x reference)
import functools

import jax
import jax.numpy as jnp
from jax import lax
from jax.experimental import pallas as pl
from jax.experimental.pallas import tpu as pltpu
from jax.experimental.pallas import tpu_sc as plsc

_D = 256
_N = 18432
_NC, _NS = 2, 16
_NW = _NC * _NS
_ROWS_W = _N // _NW
_CHUNK_ROWS = 144
_NCHUNK = _ROWS_W // _CHUNK_ROWS
_CW = _CHUNK_ROWS * _D
_L = 16
_NG = _D // _L


def _sc_body(x_hbm, w_hbm, q_hbm, part_hbm, idx_hbm,
             w0_v, x_v, q_v, idx_v, acc_v):
    wid = lax.axis_index("s") * _NC + lax.axis_index("c")
    base = wid * _ROWS_W * _D

    pltpu.sync_copy(w_hbm.at[pl.ds(0, _D)], w0_v)
    w0s = [w0_v[pl.ds(_L * j, _L)] for j in range(_NG)]

    def _fill(r, carry):
        for j in range(_NG):
            q_v[pl.ds(r * _D + _L * j, _L)] = w0s[j]
        return carry
    lax.fori_loop(0, _CHUNK_ROWS, _fill, 0)

    zi = jnp.zeros((_L,), jnp.int32)

    def _zfill(r, carry):
        idx_v[pl.ds(r * _L, _L)] = zi
        return carry
    lax.fori_loop(0, _ROWS_W // _L, _zfill, 0)
    pltpu.sync_copy(idx_v, idx_hbm.at[pl.ds(wid * _ROWS_W, _ROWS_W)])

    accs = tuple(jnp.zeros((_L,), jnp.float32) for _ in range(_NG))
    for c in range(_NCHUNK):
        off = base + c * _CW
        pltpu.sync_copy(x_hbm.at[pl.ds(off, _CW)], x_v)
        pltpu.sync_copy(q_v, q_hbm.at[pl.ds(off, _CW)])

        def _row(r, accs):
            out = []
            for j in range(_NG):
                d = x_v[pl.ds(r * _D + _L * j, _L)] - w0s[j]
                out.append(accs[j] + d * d)
            return tuple(out)
        accs = lax.fori_loop(0, _CHUNK_ROWS, _row, accs)

    acc = accs[0]
    for j in range(1, _NG):
        acc = acc + accs[j]
    acc_v[...] = acc
    pltpu.sync_copy(acc_v, part_hbm.at[pl.ds(wid * _L, _L)])


_sc_kernel = functools.partial(
    pl.kernel,
    out_type=[
        jax.ShapeDtypeStruct((_N * _D,), jnp.float32),
        jax.ShapeDtypeStruct((_NW * _L,), jnp.float32),
        jax.ShapeDtypeStruct((_N,), jnp.int32),
    ],
    mesh=plsc.VectorSubcoreMesh(core_axis_name="c", subcore_axis_name="s",
                                num_cores=_NC, num_subcores=_NS),
    scratch_types=[
        pltpu.VMEM((_D,), jnp.float32),
        pltpu.VMEM((_CW,), jnp.float32),
        pltpu.VMEM((_CW,), jnp.float32),
        pltpu.VMEM((_ROWS_W,), jnp.int32),
        pltpu.VMEM((_L,), jnp.float32),
    ],
)(_sc_body)


def _combine_body(p_ref, loss_ref, perp_ref):
    sse = jnp.sum(p_ref[...])
    loss = sse * (jnp.float32(1.25) / jnp.float32(_N * _D))
    loss_ref[...] = jnp.full((1, 1), loss, jnp.float32)
    perp = jnp.exp(-(jnp.log(jnp.float32(1.0) + jnp.float32(1e-10))))
    perp_ref[...] = jnp.full((1, 1), perp, jnp.float32)


def kernel(inputs, W):
    shape = inputs.shape
    flat = inputs.reshape(-1)
    wflat = W.reshape(-1)

    q, part, idx = _sc_kernel(flat, wflat)

    loss, perp = pl.pallas_call(
        _combine_body,
        out_shape=[
            jax.ShapeDtypeStruct((1, 1), jnp.float32),
            jax.ShapeDtypeStruct((1, 1), jnp.float32),
        ],
    )(part.reshape(_NW, _L))

    return (q.reshape(shape), loss.reshape(()), perp.reshape(()),
            idx.reshape(shape[:2]))


# speedup vs baseline: 1.4966x; 1.4966x over previous
import functools

import jax
import jax.numpy as jnp
from jax import lax
from jax.experimental import pallas as pl
from jax.experimental.pallas import tpu as pltpu
from jax.experimental.pallas import tpu_sc as plsc

_D = 256
_N = 18432
_NSC_ROWS = _N // 2
_NC, _NS = 2, 16
_NW = _NC * _NS
_ROWS_W = _NSC_ROWS // _NW
_L = 16
_NG = _D // _L

_TC_BR = 1152
_TC_STEPS = _N // _TC_BR
_TC_RED = (_N - _NSC_ROWS) // _TC_BR


def _sc_body(x_hbm, w_hbm, part_hbm, idx_hbm, w0_v, x_v, idx_v, acc_v):
    wid = lax.axis_index("s") * _NC + lax.axis_index("c")
    nwords = _ROWS_W * _D
    base = wid * nwords

    pltpu.sync_copy(w_hbm.at[pl.ds(0, _D)], w0_v)
    w0s = [w0_v[pl.ds(_L * j, _L)] for j in range(_NG)]

    zi = jnp.zeros((_L,), jnp.int32)
    idx_w = _N // _NW

    def _zfill(r, carry):
        idx_v[pl.ds(r * _L, _L)] = zi
        return carry
    lax.fori_loop(0, idx_w // _L, _zfill, 0)
    pltpu.sync_copy(idx_v, idx_hbm.at[pl.ds(wid * idx_w, idx_w)])

    pltpu.sync_copy(x_hbm.at[pl.ds(base, nwords)], x_v)
    accs = tuple(jnp.zeros((_L,), jnp.float32) for _ in range(_NG))

    def _row(r, accs):
        out = []
        for j in range(_NG):
            d = x_v[pl.ds(r * _D + _L * j, _L)] - w0s[j]
            out.append(accs[j] + d * d)
        return tuple(out)
    accs = lax.fori_loop(0, _ROWS_W, _row, accs)

    acc = accs[0]
    for j in range(1, _NG):
        acc = acc + accs[j]
    acc_v[...] = acc
    pltpu.sync_copy(acc_v, part_hbm.at[pl.ds(wid * _L, _L)])


_sc_kernel = functools.partial(
    pl.kernel,
    out_type=[
        jax.ShapeDtypeStruct((_NW * _L,), jnp.float32),
        jax.ShapeDtypeStruct((_N,), jnp.int32),
    ],
    mesh=plsc.VectorSubcoreMesh(core_axis_name="c", subcore_axis_name="s",
                                num_cores=_NC, num_subcores=_NS),
    scratch_types=[
        pltpu.VMEM((_D,), jnp.float32),
        pltpu.VMEM((_ROWS_W * _D,), jnp.float32),
        pltpu.VMEM((_N // _NW,), jnp.int32),
        pltpu.VMEM((_L,), jnp.float32),
    ],
)(_sc_body)


def _tc_body(x_ref, w_ref, q_ref, sse_ref, acc_ref):
    i = pl.program_id(0)
    w0 = w_ref[0:1, :]

    @pl.when(i < _TC_RED)
    def _reduce_and_write():
        x = x_ref[...]
        d = w0 - x
        q_ref[...] = x + d
        part = jnp.sum(d * d)

        @pl.when(i == 0)
        def _init():
            acc_ref[0, 0] = part

        @pl.when(i > 0)
        def _acc():
            acc_ref[0, 0] += part

    @pl.when(i >= _TC_RED)
    def _write_only():
        q_ref[...] = jnp.broadcast_to(w0, (_TC_BR, _D))

    @pl.when(i == _TC_STEPS - 1)
    def _fin():
        sse_ref[...] = jnp.full((1, 1), acc_ref[0, 0], jnp.float32)


def _combine_body(p_ref, sse_ref, loss_ref, perp_ref):
    sse = jnp.sum(p_ref[...]) + sse_ref[0, 0]
    loss = sse * (jnp.float32(1.25) / jnp.float32(_N * _D))
    loss_ref[...] = jnp.full((1, 1), loss, jnp.float32)
    perp = jnp.exp(-(jnp.log(jnp.float32(1.0) + jnp.float32(1e-10))))
    perp_ref[...] = jnp.full((1, 1), perp, jnp.float32)


def kernel(inputs, W):
    shape = inputs.shape
    flat = inputs.reshape(-1, _D)
    x_sc = flat[_N - _NSC_ROWS:].reshape(-1)
    wflat = W.reshape(-1)

    part, idx = _sc_kernel(x_sc, wflat)

    q, sse = pl.pallas_call(
        _tc_body,
        grid=(_TC_STEPS,),
        in_specs=[
            pl.BlockSpec((_TC_BR, _D), lambda i: (jnp.minimum(i, _TC_RED - 1), 0)),
            pl.BlockSpec((8, _D), lambda i: (0, 0)),
        ],
        out_specs=[
            pl.BlockSpec((_TC_BR, _D), lambda i: (i, 0)),
            pl.BlockSpec((1, 1), lambda i: (0, 0)),
        ],
        out_shape=[
            jax.ShapeDtypeStruct((_N, _D), jnp.float32),
            jax.ShapeDtypeStruct((1, 1), jnp.float32),
        ],
        scratch_shapes=[pltpu.SMEM((1, 1), jnp.float32)],
    )(flat, W)

    loss, perp = pl.pallas_call(
        _combine_body,
        out_shape=[
            jax.ShapeDtypeStruct((1, 1), jnp.float32),
            jax.ShapeDtypeStruct((1, 1), jnp.float32),
        ],
    )(part.reshape(_NW, _L), sse)

    return (q.reshape(shape), loss.reshape(()), perp.reshape(()),
            idx.reshape(shape[:2]))


# speedup vs baseline: 2.6169x; 1.7485x over previous
import functools

import jax
import jax.numpy as jnp
from jax import lax
from jax.experimental import pallas as pl
from jax.experimental.pallas import tpu as pltpu
from jax.experimental.pallas import tpu_sc as plsc

_D = 256
_N = 18432
_NSC_ROWS = _N // 2
_SC_ROW0 = _N - _NSC_ROWS
_NC, _NS = 2, 16
_NW = _NC * _NS
_ROWS_W = _NSC_ROWS // _NW
_L = 16
_NG = _D // _L

_TC_BR = 1152
_TC_STEPS = _N // _TC_BR
_TC_RED = (_N - _NSC_ROWS) // _TC_BR


def _sc_body(x_hbm, w_hbm, part_hbm, idx_hbm, w0_v, x_v, idx_v, acc_v):
    wid = lax.axis_index("s") * _NC + lax.axis_index("c")
    row0 = _SC_ROW0 + wid * _ROWS_W

    pltpu.sync_copy(w_hbm.at[0], w0_v)
    w0s = [w0_v[pl.ds(_L * j, _L)] for j in range(_NG)]

    zi = jnp.zeros((_L,), jnp.int32)

    def _zfill(r, carry):
        idx_v[pl.ds(r * _L, _L)] = zi
        return carry
    lax.fori_loop(0, (_N // _NW) // _L, _zfill, 0)
    pltpu.sync_copy(idx_v, idx_hbm.at[pl.ds(wid * (_N // _NW), _N // _NW)])

    pltpu.sync_copy(x_hbm.at[pl.ds(row0, _ROWS_W), :], x_v)
    accs = tuple(jnp.zeros((_L,), jnp.float32) for _ in range(_NG))

    def _row(r, accs):
        out = []
        for j in range(_NG):
            d = x_v[r, pl.ds(_L * j, _L)] - w0s[j]
            out.append(accs[j] + d * d)
        return tuple(out)
    accs = lax.fori_loop(0, _ROWS_W, _row, accs)

    acc = accs[0]
    for j in range(1, _NG):
        acc = acc + accs[j]
    acc_v[...] = acc
    pltpu.sync_copy(acc_v, part_hbm.at[wid])


_sc_kernel = functools.partial(
    pl.kernel,
    out_type=[
        jax.ShapeDtypeStruct((_NW, _L), jnp.float32),
        jax.ShapeDtypeStruct((_N,), jnp.int32),
    ],
    mesh=plsc.VectorSubcoreMesh(core_axis_name="c", subcore_axis_name="s",
                                num_cores=_NC, num_subcores=_NS),
    scratch_types=[
        pltpu.VMEM((_D,), jnp.float32),
        pltpu.VMEM((_ROWS_W, _D), jnp.float32),
        pltpu.VMEM((_N // _NW,), jnp.int32),
        pltpu.VMEM((_L,), jnp.float32),
    ],
)(_sc_body)


def _tc_body(x_ref, w_ref, q_ref, sse_ref, acc_ref):
    i = pl.program_id(0)
    w0 = w_ref[0:1, :]

    @pl.when(i < _TC_RED)
    def _reduce_and_write():
        x = x_ref[...]
        d = w0 - x
        q_ref[...] = x + d
        part = jnp.sum(d * d)

        @pl.when(i == 0)
        def _init():
            acc_ref[0, 0] = part

        @pl.when(i > 0)
        def _acc():
            acc_ref[0, 0] += part

    @pl.when(i >= _TC_RED)
    def _write_only():
        q_ref[...] = jnp.broadcast_to(w0, (_TC_BR, _D))

    @pl.when(i == _TC_STEPS - 1)
    def _fin():
        sse_ref[...] = jnp.full((1, 1), acc_ref[0, 0], jnp.float32)


def _combine_body(p_ref, sse_ref, loss_ref, perp_ref):
    sse = jnp.sum(p_ref[...]) + sse_ref[0, 0]
    loss = sse * (jnp.float32(1.25) / jnp.float32(_N * _D))
    loss_ref[...] = jnp.full((1, 1), loss, jnp.float32)
    perp = jnp.exp(-(jnp.log(jnp.float32(1.0) + jnp.float32(1e-10))))
    perp_ref[...] = jnp.full((1, 1), perp, jnp.float32)


def kernel(inputs, W):
    shape = inputs.shape
    flat = inputs.reshape(-1, _D)

    part, idx = _sc_kernel(flat, W)

    q, sse = pl.pallas_call(
        _tc_body,
        grid=(_TC_STEPS,),
        in_specs=[
            pl.BlockSpec((_TC_BR, _D), lambda i: (jnp.minimum(i, _TC_RED - 1), 0)),
            pl.BlockSpec((8, _D), lambda i: (0, 0)),
        ],
        out_specs=[
            pl.BlockSpec((_TC_BR, _D), lambda i: (i, 0)),
            pl.BlockSpec((1, 1), lambda i: (0, 0)),
        ],
        out_shape=[
            jax.ShapeDtypeStruct((_N, _D), jnp.float32),
            jax.ShapeDtypeStruct((1, 1), jnp.float32),
        ],
        scratch_shapes=[pltpu.SMEM((1, 1), jnp.float32)],
    )(flat, W)

    loss, perp = pl.pallas_call(
        _combine_body,
        out_shape=[
            jax.ShapeDtypeStruct((1, 1), jnp.float32),
            jax.ShapeDtypeStruct((1, 1), jnp.float32),
        ],
    )(part, sse)

    return (q.reshape(shape), loss.reshape(()), perp.reshape(()),
            idx.reshape(shape[:2]))


# speedup vs baseline: 5.8912x; 2.2512x over previous
import jax
import jax.numpy as jnp
from jax.experimental import pallas as pl
from jax.experimental.pallas import tpu as pltpu

_D = 256
_BR = 4608


def _vq_body(x_ref, w_ref, q_ref, loss_ref, perp_ref, idx_ref, acc_ref):
    i = pl.program_id(0)
    nsteps = pl.num_programs(0)
    w0 = w_ref[0:1, :]
    x = x_ref[...]
    d = w0 - x
    q_ref[...] = x + d
    part = jnp.sum(d * d)

    @pl.when(i == 0)
    def _init():
        acc_ref[0, 0] = part
        idx_ref[...] = jnp.zeros_like(idx_ref)
        perp = jnp.exp(-(jnp.log(jnp.float32(1.0) + jnp.float32(1e-10))))
        perp_ref[...] = jnp.full((1, 1), perp, jnp.float32)

    @pl.when(i > 0)
    def _acc():
        acc_ref[0, 0] += part

    @pl.when(i == nsteps - 1)
    def _fin():
        total = jnp.float32(nsteps * _BR * _D)
        loss = acc_ref[0, 0] * (jnp.float32(1.25) / total)
        loss_ref[...] = jnp.full((1, 1), loss, jnp.float32)


def kernel(inputs, W):
    shape = inputs.shape
    flat = inputs.reshape(-1, _D)
    n = flat.shape[0]
    grid = n // _BR

    q, loss, perp, idx = pl.pallas_call(
        _vq_body,
        grid=(grid,),
        in_specs=[
            pl.BlockSpec((_BR, _D), lambda i: (i, 0)),
            pl.BlockSpec((8, _D), lambda i: (0, 0)),
        ],
        out_specs=[
            pl.BlockSpec((_BR, _D), lambda i: (i, 0)),
            pl.BlockSpec((1, 1), lambda i: (0, 0)),
            pl.BlockSpec((1, 1), lambda i: (0, 0)),
            pl.BlockSpec(shape[:2], lambda i: (0, 0)),
        ],
        out_shape=[
            jax.ShapeDtypeStruct((n, _D), jnp.float32),
            jax.ShapeDtypeStruct((1, 1), jnp.float32),
            jax.ShapeDtypeStruct((1, 1), jnp.float32),
            jax.ShapeDtypeStruct(shape[:2], jnp.int32),
        ],
        scratch_shapes=[pltpu.SMEM((1, 1), jnp.float32)],
    )(flat, W)

    return (q.reshape(shape), loss.reshape(()), perp.reshape(()), idx)


# speedup vs baseline: 5.9761x; 1.0144x over previous
import jax
import jax.numpy as jnp
from jax.experimental import pallas as pl
from jax.experimental.pallas import tpu as pltpu

_D = 256
_BR = 9216


def _vq_body(x_ref, w_ref, q_ref, loss_ref, perp_ref, idx_ref, acc_ref):
    i = pl.program_id(0)
    nsteps = pl.num_programs(0)
    w0 = w_ref[0:1, :]
    x = x_ref[...]
    d = w0 - x
    q_ref[...] = x + d
    part = jnp.sum(d * d)

    @pl.when(i == 0)
    def _init():
        acc_ref[0, 0] = part
        idx_ref[...] = jnp.zeros_like(idx_ref)
        perp = jnp.exp(-(jnp.log(jnp.float32(1.0) + jnp.float32(1e-10))))
        perp_ref[...] = jnp.full((1, 1), perp, jnp.float32)

    @pl.when(i > 0)
    def _acc():
        acc_ref[0, 0] += part

    @pl.when(i == nsteps - 1)
    def _fin():
        total = jnp.float32(nsteps * _BR * _D)
        loss = acc_ref[0, 0] * (jnp.float32(1.25) / total)
        loss_ref[...] = jnp.full((1, 1), loss, jnp.float32)


def kernel(inputs, W):
    shape = inputs.shape
    flat = inputs.reshape(-1, _D)
    n = flat.shape[0]
    grid = n // _BR

    q, loss, perp, idx = pl.pallas_call(
        _vq_body,
        grid=(grid,),
        in_specs=[
            pl.BlockSpec((_BR, _D), lambda i: (i, 0)),
            pl.BlockSpec((8, _D), lambda i: (0, 0)),
        ],
        out_specs=[
            pl.BlockSpec((_BR, _D), lambda i: (i, 0)),
            pl.BlockSpec((1, 1), lambda i: (0, 0)),
            pl.BlockSpec((1, 1), lambda i: (0, 0)),
            pl.BlockSpec(shape[:2], lambda i: (0, 0)),
        ],
        out_shape=[
            jax.ShapeDtypeStruct((n, _D), jnp.float32),
            jax.ShapeDtypeStruct((1, 1), jnp.float32),
            jax.ShapeDtypeStruct((1, 1), jnp.float32),
            jax.ShapeDtypeStruct(shape[:2], jnp.int32),
        ],
        scratch_shapes=[pltpu.SMEM((1, 1), jnp.float32)],
    )(flat, W)

    return (q.reshape(shape), loss.reshape(()), perp.reshape(()), idx)
